# Initial kernel scaffold; baseline (speedup 1.0000x reference)
import jax
import jax.numpy as jnp
from jax.experimental import pallas as pl


def kernel(in_feat, edge_index, edge_weights, W0, al0, ar0, b0, W1, al1, ar1, b1):
    raise NotImplementedError("write your pallas kernel here")


# speedup vs baseline: 23.1922x; 23.1922x over previous
import functools

import jax
import jax.numpy as jnp
from jax import lax
from jax.experimental import pallas as pl
from jax.experimental.pallas import tpu as pltpu
from jax.experimental.pallas import tpu_sc as plsc

N = 10000
E = 320000
F = 128
H = 8
D = 16

NC = 2
NS = 16
NW = NC * NS
EPW = E // NW
BATCH = 80
NBATCH = EPW // BATCH
NP = 10240
RPT = NP // NS
RCH = 64
NCH = RPT // RCH

_MESH = plsc.VectorSubcoreMesh(core_axis_name="c", subcore_axis_name="s")
_LANE8 = None


def _lrelu(x):
    return jnp.maximum(x, 0.2 * x)


def _splat(vec, h):
    idx = jnp.full((16,), h, jnp.int32)
    return lax.gather(
        vec, idx[:, None],
        dimension_numbers=lax.GatherDimensionNumbers(
            offset_dims=(), collapsed_slice_dims=(0,), start_index_map=(0,)),
        slice_sizes=(1,), mode=lax.GatherScatterMode.PROMISE_IN_BOUNDS)


def _zero_rows(buf, nrows, ncols16):
    z = jnp.zeros((16,), jnp.float32)

    def body(r, _):
        for c in range(ncols16):
            buf[r, pl.ds(c * 16, 16)] = z
        return 0

    lax.fori_loop(0, nrows, body, 0, unroll=False)


def _wid_base(cid, sid):
    return (cid * NS + sid) * EPW


def _sc_weighted_agg(h, src, dst, w):
    def body(h_hbm, src_hbm, dst_hbm, w_hbm, out_hbm,
             sidx, didx, wv, rows, orows, fbuf, accum, sem):
        cid = lax.axis_index("c")
        sid = lax.axis_index("s")

        _zero_rows(fbuf, RCH, F // 16)
        for k in range(NCH):
            r0 = sid * RPT + k * RCH
            pltpu.sync_copy(fbuf, accum.at[pl.ds(r0, RCH)])
        plsc.subcore_barrier()

        base0 = _wid_base(cid, sid)

        def batch(b, _):
            base = base0 + b * BATCH
            pltpu.sync_copy(src_hbm.at[pl.ds(base, BATCH)], sidx)
            pltpu.sync_copy(dst_hbm.at[pl.ds(base, BATCH)], didx)
            pltpu.sync_copy(w_hbm.at[pl.ds(base, BATCH)], wv)
            pltpu.async_copy(h_hbm.at[sidx], rows, sem).wait()

            def edge(i, _):
                g16 = pl.multiple_of((i // 16) * 16, 16)
                wvec = wv[pl.ds(g16, 16)]
                ws = _splat(wvec, i % 16)
                for c in range(F // 16):
                    orows[i, pl.ds(c * 16, 16)] = rows[i, pl.ds(c * 16, 16)] * ws
                return 0

            lax.fori_loop(0, BATCH, edge, 0, unroll=False)
            pltpu.sync_copy(orows, accum.at[didx], add=True)
            return 0

        lax.fori_loop(0, NBATCH, batch, 0, unroll=False)

        plsc.subcore_barrier()
        for k in range(NCH):
            r0 = sid * RPT + k * RCH
            pltpu.sync_copy(accum.at[pl.ds(r0, RCH)], fbuf)
            pltpu.sync_copy(fbuf, out_hbm.at[pl.ds(cid * NP + r0, RCH)])

    f = pl.kernel(
        body,
        out_type=jax.ShapeDtypeStruct((NC * NP, F), jnp.float32),
        mesh=_MESH,
        compiler_params=pltpu.CompilerParams(use_tc_tiling_on_sc=False),
        scratch_types=[
            pltpu.VMEM((BATCH,), jnp.int32),
            pltpu.VMEM((BATCH,), jnp.int32),
            pltpu.VMEM((BATCH,), jnp.float32),
            pltpu.VMEM((BATCH, F), jnp.float32),
            pltpu.VMEM((BATCH, F), jnp.float32),
            pltpu.VMEM((RCH, F), jnp.float32),
            pltpu.VMEM_SHARED((NP, F), jnp.float32),
            pltpu.SemaphoreType.DMA,
        ],
    )
    return f(h, src, dst, w)


def _sc_attn_msg(feat, elp, erp, gmax, src, dst):
    def body(feat_hbm, elp_hbm, erp_hbm, gmax_hbm, src_hbm, dst_hbm,
             msg_hbm, den_hbm,
             sidx, didx, frows, elrows, errows, orows, drows,
             fbuf, dfbuf, gmv, maccum, daccum, sem):
        cid = lax.axis_index("c")
        sid = lax.axis_index("s")

        pltpu.sync_copy(gmax_hbm, gmv)
        lane = lax.iota(jnp.int32, 16)
        lane8 = lane < 8

        _zero_rows(fbuf, RCH, F // 16)
        _zero_rows(dfbuf, RCH, 1)
        for k in range(NCH):
            r0 = sid * RPT + k * RCH
            pltpu.sync_copy(fbuf, maccum.at[pl.ds(r0, RCH)])
            pltpu.sync_copy(dfbuf, daccum.at[pl.ds(r0, RCH)])
        plsc.subcore_barrier()

        gm = gmv[0, pl.ds(0, 16)]
        base0 = _wid_base(cid, sid)

        def batch(b, _):
            base = base0 + b * BATCH
            pltpu.sync_copy(src_hbm.at[pl.ds(base, BATCH)], sidx)
            pltpu.sync_copy(dst_hbm.at[pl.ds(base, BATCH)], didx)
            pltpu.async_copy(feat_hbm.at[sidx], frows, sem).wait()
            pltpu.async_copy(elp_hbm.at[sidx], elrows, sem).wait()
            pltpu.async_copy(erp_hbm.at[didx], errows, sem).wait()

            def edge(i, _):
                el = elrows[i, pl.ds(0, 16)]
                er = errows[i, pl.ds(0, 16)]
                ex = jnp.exp(_lrelu(el + er) - _lrelu(gm + er))
                ex = jnp.where(lane8, ex, 0.0)
                drows[i, pl.ds(0, 16)] = ex
                for hh in range(H):
                    exh = _splat(ex, hh)
                    orows[i, pl.ds(hh * 16, 16)] = (
                        frows[i, pl.ds(hh * 16, 16)] * exh)
                return 0

            lax.fori_loop(0, BATCH, edge, 0, unroll=False)
            pltpu.sync_copy(orows, maccum.at[didx], add=True)
            pltpu.sync_copy(drows, daccum.at[didx], add=True)
            return 0

        lax.fori_loop(0, NBATCH, batch, 0, unroll=False)

        plsc.subcore_barrier()
        for k in range(NCH):
            r0 = sid * RPT + k * RCH
            pltpu.sync_copy(maccum.at[pl.ds(r0, RCH)], fbuf)
            pltpu.sync_copy(fbuf, msg_hbm.at[pl.ds(cid * NP + r0, RCH)])
            pltpu.sync_copy(daccum.at[pl.ds(r0, RCH)], dfbuf)
            pltpu.sync_copy(dfbuf, den_hbm.at[pl.ds(cid * NP + r0, RCH)])

    f = pl.kernel(
        body,
        out_type=[jax.ShapeDtypeStruct((NC * NP, F), jnp.float32),
                  jax.ShapeDtypeStruct((NC * NP, 16), jnp.float32)],
        mesh=_MESH,
        compiler_params=pltpu.CompilerParams(use_tc_tiling_on_sc=False),
        scratch_types=[
            pltpu.VMEM((BATCH,), jnp.int32),
            pltpu.VMEM((BATCH,), jnp.int32),
            pltpu.VMEM((BATCH, F), jnp.float32),
            pltpu.VMEM((BATCH, 16), jnp.float32),
            pltpu.VMEM((BATCH, 16), jnp.float32),
            pltpu.VMEM((BATCH, F), jnp.float32),
            pltpu.VMEM((BATCH, 16), jnp.float32),
            pltpu.VMEM((RCH, F), jnp.float32),
            pltpu.VMEM((RCH, 16), jnp.float32),
            pltpu.VMEM((1, 16), jnp.float32),
            pltpu.VMEM_SHARED((NP, F), jnp.float32),
            pltpu.VMEM_SHARED((NP, 16), jnp.float32),
            pltpu.SemaphoreType.DMA,
        ],
    )
    return f(feat, elp, erp, gmax, src, dst)


_RB = 1024
_NBLK = NP // _RB

_DOT = functools.partial(jnp.dot, preferred_element_type=jnp.float32,
                         precision=lax.Precision.HIGHEST)


def _tc_feat(hbase, agg2, W, Al16, Ar16):
    def body(h_ref, a0_ref, a1_ref, w_ref, al_ref, ar_ref,
             feat_ref, elp_ref, erp_ref, gmax_ref):
        h2 = h_ref[...] + a0_ref[...] + a1_ref[...]
        feat = _DOT(h2, w_ref[...])
        feat_ref[...] = feat
        elp = _DOT(feat, al_ref[...])
        erp = _DOT(feat, ar_ref[...])
        elp_ref[...] = elp
        erp_ref[...] = erp

        @pl.when(pl.program_id(0) == 0)
        def _():
            gmax_ref[...] = jnp.full((1, 16), -1e30, jnp.float32)

        gmax_ref[...] = jnp.maximum(gmax_ref[...],
                                    jnp.max(elp, axis=0, keepdims=True))

    return pl.pallas_call(
        body,
        grid=(_NBLK,),
        in_specs=[
            pl.BlockSpec((_RB, F), lambda g: (g, 0)),
            pl.BlockSpec((_RB, F), lambda g: (g, 0)),
            pl.BlockSpec((_RB, F), lambda g: (g + _NBLK, 0)),
            pl.BlockSpec((F, F), lambda g: (0, 0)),
            pl.BlockSpec((F, 16), lambda g: (0, 0)),
            pl.BlockSpec((F, 16), lambda g: (0, 0)),
        ],
        out_specs=[
            pl.BlockSpec((_RB, F), lambda g: (g, 0)),
            pl.BlockSpec((_RB, 16), lambda g: (g, 0)),
            pl.BlockSpec((_RB, 16), lambda g: (g, 0)),
            pl.BlockSpec((1, 16), lambda g: (0, 0)),
        ],
        out_shape=[
            jax.ShapeDtypeStruct((NP, F), jnp.float32),
            jax.ShapeDtypeStruct((NP, 16), jnp.float32),
            jax.ShapeDtypeStruct((NP, 16), jnp.float32),
            jax.ShapeDtypeStruct((1, 16), jnp.float32),
        ],
    )(hbase, agg2, agg2, W, Al16, Ar16)


def _tc_epilogue(msg2, den2, Rmat, bias, base, residual_relu):
    def body(m0_ref, m1_ref, d0_ref, d1_ref, r_ref, b_ref, base_ref, o_ref):
        den = d0_ref[...] + d1_ref[...]
        den_bc = _DOT(den[:, 0:8], r_ref[...])
        out = (m0_ref[...] + m1_ref[...]) / jnp.maximum(den_bc, 1e-30)
        out = out + b_ref[...]
        if residual_relu:
            out = jnp.maximum(out + base_ref[...], 0.0)
        o_ref[...] = out

    return pl.pallas_call(
        body,
        grid=(_NBLK,),
        in_specs=[
            pl.BlockSpec((_RB, F), lambda g: (g, 0)),
            pl.BlockSpec((_RB, F), lambda g: (g + _NBLK, 0)),
            pl.BlockSpec((_RB, 16), lambda g: (g, 0)),
            pl.BlockSpec((_RB, 16), lambda g: (g + _NBLK, 0)),
            pl.BlockSpec((8, F), lambda g: (0, 0)),
            pl.BlockSpec((1, F), lambda g: (0, 0)),
            pl.BlockSpec((_RB, F), lambda g: (g, 0)),
        ],
        out_specs=pl.BlockSpec((_RB, F), lambda g: (g, 0)),
        out_shape=jax.ShapeDtypeStruct((NP, F), jnp.float32),
    )(msg2, msg2, den2, den2, Rmat, bias, base)


def _attn_mat(a):
    m = jnp.zeros((H, D, 16), jnp.float32)
    m = m + a.reshape(H, D, 1) * jax.nn.one_hot(jnp.arange(H), 16,
                                               dtype=jnp.float32)[:, None, :]
    return m.reshape(F, 16)


def kernel(in_feat, edge_index, edge_weights, W0, al0, ar0, b0, W1, al1, ar1, b1):
    src = edge_index[0].astype(jnp.int32)
    dst = edge_index[1].astype(jnp.int32)
    w = edge_weights.astype(jnp.float32)

    Al0, Ar0 = _attn_mat(al0), _attn_mat(ar0)
    Al1, Ar1 = _attn_mat(al1), _attn_mat(ar1)
    Rmat = jnp.kron(jnp.eye(8, dtype=jnp.float32), jnp.ones((1, 16), jnp.float32))
    b0r = b0.reshape(1, F)
    b1r = b1.reshape(1, F)
    hp = jnp.pad(in_feat, ((0, NP - N), (0, 0)))

    agg0 = _sc_weighted_agg(hp, src, dst, w)
    feat0, elp0, erp0, gmax0 = _tc_feat(hp, agg0, W0, Al0, Ar0)
    msg0, den0 = _sc_attn_msg(feat0, elp0, erp0, gmax0, src, dst)
    h = _tc_epilogue(msg0, den0, Rmat, b0r, hp, True)

    agg1 = _sc_weighted_agg(h, src, dst, w)
    feat1, elp1, erp1, gmax1 = _tc_feat(h, agg1, W1, Al1, Ar1)
    msg1, den1 = _sc_attn_msg(feat1, elp1, erp1, gmax1, src, dst)
    out = _tc_epilogue(msg1, den1, Rmat, b1r, h, False)
    return out[:N]


# speedup vs baseline: 69.1215x; 2.9804x over previous
import functools

import jax
import jax.numpy as jnp
from jax import lax
from jax.experimental import pallas as pl
from jax.experimental.pallas import tpu as pltpu
from jax.experimental.pallas import tpu_sc as plsc

N = 10000
E = 320000
F = 128
H = 8
D = 16

NC = 2
NS = 16
NW = NC * NS
EPW = E // NW
BATCH = 80
NBATCH = EPW // BATCH
NP = 10240
RPT = NP // NS
RCH = 64
NCH = RPT // RCH

_MESH = plsc.VectorSubcoreMesh(core_axis_name="c", subcore_axis_name="s")
_LANE8 = None


def _lrelu(x):
    return jnp.maximum(x, 0.2 * x)


def _splat(vec, h):
    idx = jnp.full((16,), h, jnp.int32)
    return lax.gather(
        vec, idx[:, None],
        dimension_numbers=lax.GatherDimensionNumbers(
            offset_dims=(), collapsed_slice_dims=(0,), start_index_map=(0,)),
        slice_sizes=(1,), mode=lax.GatherScatterMode.PROMISE_IN_BOUNDS)


def _zero_rows(buf, nrows, ncols16):
    z = jnp.zeros((16,), jnp.float32)

    def body(r, _):
        for c in range(ncols16):
            buf[r, pl.ds(c * 16, 16)] = z
        return 0

    lax.fori_loop(0, nrows, body, 0, unroll=False)


def _wid_base(cid, sid):
    return (cid * NS + sid) * EPW


def _sc_weighted_agg(h, src, dst, w, zf):
    def body(h_hbm, src_hbm, dst_hbm, w_hbm, zf_hbm, out_hbm,
             sidx0, sidx1, didx0, didx1, wv0, wv1, rows0, rows1,
             orows, accum, sems):
        cid = lax.axis_index("c")
        sid = lax.axis_index("s")
        sidx = (sidx0, sidx1)
        didx = (didx0, didx1)
        wv = (wv0, wv1)
        rows = (rows0, rows1)

        stripe = pl.ds(sid * RPT, RPT)
        pltpu.sync_copy(zf_hbm.at[stripe], accum.at[stripe])
        plsc.subcore_barrier()

        base0 = _wid_base(cid, sid)

        def load_idx(b, s):
            base = base0 + b * BATCH
            pltpu.sync_copy(src_hbm.at[pl.ds(base, BATCH)], sidx[s])
            pltpu.sync_copy(dst_hbm.at[pl.ds(base, BATCH)], didx[s])
            pltpu.sync_copy(w_hbm.at[pl.ds(base, BATCH)], wv[s])

        def start_g(s):
            pltpu.async_copy(h_hbm.at[sidx[s]], rows[s], sems.at[s])

        def wait_g(s):
            pltpu.make_async_copy(h_hbm.at[sidx[s]], rows[s], sems.at[s]).wait()

        load_idx(0, 0)
        start_g(0)

        def pair(b2, _):
            for s in range(2):
                b = b2 + s

                @pl.when(b < NBATCH)
                def _():
                    nb = b + 1

                    @pl.when(nb < NBATCH)
                    def _():
                        load_idx(nb, 1 - s)
                        start_g(1 - s)

                    wait_g(s)

                    @plsc.parallel_loop(0, BATCH, unroll=4)
                    def _(i):
                        g16 = pl.multiple_of((i // 16) * 16, 16)
                        ws = _splat(wv[s][pl.ds(g16, 16)], i % 16)
                        for c in range(F // 16):
                            orows[i, pl.ds(c * 16, 16)] = (
                                rows[s][i, pl.ds(c * 16, 16)] * ws)

                    pltpu.sync_copy(orows, accum.at[didx[s]], add=True)

            return 0

        lax.fori_loop(0, (NBATCH + 1) // 2, lambda j, c: pair(j * 2, c), 0,
                      unroll=False)

        plsc.subcore_barrier()
        pltpu.sync_copy(accum.at[stripe],
                        out_hbm.at[pl.ds(cid * NP + sid * RPT, RPT)])

    f = pl.kernel(
        body,
        out_type=jax.ShapeDtypeStruct((NC * NP, F), jnp.float32),
        mesh=_MESH,
        compiler_params=pltpu.CompilerParams(use_tc_tiling_on_sc=False),
        scratch_types=[
            pltpu.VMEM((BATCH,), jnp.int32),
            pltpu.VMEM((BATCH,), jnp.int32),
            pltpu.VMEM((BATCH,), jnp.int32),
            pltpu.VMEM((BATCH,), jnp.int32),
            pltpu.VMEM((BATCH,), jnp.float32),
            pltpu.VMEM((BATCH,), jnp.float32),
            pltpu.VMEM((BATCH, F), jnp.float32),
            pltpu.VMEM((BATCH, F), jnp.float32),
            pltpu.VMEM((BATCH, F), jnp.float32),
            pltpu.VMEM_SHARED((NP, F), jnp.float32),
            pltpu.SemaphoreType.DMA((2,)),
        ],
    )
    return f(h, src, dst, w, zf)


def _sc_attn_msg(feat, elp, erp, gmax, src, dst, zf, zd):
    def body(feat_hbm, elp_hbm, erp_hbm, gmax_hbm, src_hbm, dst_hbm,
             zf_hbm, zd_hbm, msg_hbm, den_hbm,
             sidx0, sidx1, didx0, didx1, frows0, frows1,
             elrows0, elrows1, errows0, errows1, orows, drows,
             gmv, maccum, daccum, sems):
        cid = lax.axis_index("c")
        sid = lax.axis_index("s")
        sidx = (sidx0, sidx1)
        didx = (didx0, didx1)
        frows = (frows0, frows1)
        elrows = (elrows0, elrows1)
        errows = (errows0, errows1)

        pltpu.sync_copy(gmax_hbm, gmv)
        lane = lax.iota(jnp.int32, 16)
        lane8 = lane < 8

        stripe = pl.ds(sid * RPT, RPT)
        pltpu.sync_copy(zf_hbm.at[stripe], maccum.at[stripe])
        pltpu.sync_copy(zd_hbm.at[stripe], daccum.at[stripe])
        plsc.subcore_barrier()

        gm = gmv[0, pl.ds(0, 16)]
        base0 = _wid_base(cid, sid)

        def load_idx(b, s):
            base = base0 + b * BATCH
            pltpu.sync_copy(src_hbm.at[pl.ds(base, BATCH)], sidx[s])
            pltpu.sync_copy(dst_hbm.at[pl.ds(base, BATCH)], didx[s])

        def start_g(s):
            pltpu.async_copy(feat_hbm.at[sidx[s]], frows[s], sems.at[s])
            pltpu.async_copy(elp_hbm.at[sidx[s]], elrows[s], sems.at[s])
            pltpu.async_copy(erp_hbm.at[didx[s]], errows[s], sems.at[s])

        def wait_g(s):
            pltpu.make_async_copy(feat_hbm.at[sidx[s]], frows[s], sems.at[s]).wait()
            pltpu.make_async_copy(elp_hbm.at[sidx[s]], elrows[s], sems.at[s]).wait()
            pltpu.make_async_copy(erp_hbm.at[didx[s]], errows[s], sems.at[s]).wait()

        load_idx(0, 0)
        start_g(0)

        def pair(b2, _):
            for s in range(2):
                b = b2 + s

                @pl.when(b < NBATCH)
                def _():
                    nb = b + 1

                    @pl.when(nb < NBATCH)
                    def _():
                        load_idx(nb, 1 - s)
                        start_g(1 - s)

                    wait_g(s)

                    @plsc.parallel_loop(0, BATCH, unroll=4)
                    def _(i):
                        el = elrows[s][i, pl.ds(0, 16)]
                        er = errows[s][i, pl.ds(0, 16)]
                        ex = jnp.exp(_lrelu(el + er) - _lrelu(gm + er))
                        ex = jnp.where(lane8, ex, 0.0)
                        drows[i, pl.ds(0, 16)] = ex
                        for hh in range(H):
                            exh = _splat(ex, hh)
                            orows[i, pl.ds(hh * 16, 16)] = (
                                frows[s][i, pl.ds(hh * 16, 16)] * exh)

                    pltpu.sync_copy(orows, maccum.at[didx[s]], add=True)
                    pltpu.sync_copy(drows, daccum.at[didx[s]], add=True)

            return 0

        lax.fori_loop(0, (NBATCH + 1) // 2, lambda j, c: pair(j * 2, c), 0,
                      unroll=False)

        plsc.subcore_barrier()
        pltpu.sync_copy(maccum.at[stripe],
                        msg_hbm.at[pl.ds(cid * NP + sid * RPT, RPT)])
        pltpu.sync_copy(daccum.at[stripe],
                        den_hbm.at[pl.ds(cid * NP + sid * RPT, RPT)])

    f = pl.kernel(
        body,
        out_type=[jax.ShapeDtypeStruct((NC * NP, F), jnp.float32),
                  jax.ShapeDtypeStruct((NC * NP, 16), jnp.float32)],
        mesh=_MESH,
        compiler_params=pltpu.CompilerParams(use_tc_tiling_on_sc=False),
        scratch_types=[
            pltpu.VMEM((BATCH,), jnp.int32),
            pltpu.VMEM((BATCH,), jnp.int32),
            pltpu.VMEM((BATCH,), jnp.int32),
            pltpu.VMEM((BATCH,), jnp.int32),
            pltpu.VMEM((BATCH, F), jnp.float32),
            pltpu.VMEM((BATCH, F), jnp.float32),
            pltpu.VMEM((BATCH, 16), jnp.float32),
            pltpu.VMEM((BATCH, 16), jnp.float32),
            pltpu.VMEM((BATCH, 16), jnp.float32),
            pltpu.VMEM((BATCH, 16), jnp.float32),
            pltpu.VMEM((BATCH, F), jnp.float32),
            pltpu.VMEM((BATCH, 16), jnp.float32),
            pltpu.VMEM((1, 16), jnp.float32),
            pltpu.VMEM_SHARED((NP, F), jnp.float32),
            pltpu.VMEM_SHARED((NP, 16), jnp.float32),
            pltpu.SemaphoreType.DMA((2,)),
        ],
    )
    return f(feat, elp, erp, gmax, src, dst, zf, zd)


_RB = 1024
_NBLK = NP // _RB

_DOT = functools.partial(jnp.dot, preferred_element_type=jnp.float32,
                         precision=lax.Precision.HIGHEST)


def _tc_feat(hbase, agg2, W, Al16, Ar16):
    def body(h_ref, a0_ref, a1_ref, w_ref, al_ref, ar_ref,
             feat_ref, elp_ref, erp_ref, gmax_ref):
        h2 = h_ref[...] + a0_ref[...] + a1_ref[...]
        feat = _DOT(h2, w_ref[...])
        feat_ref[...] = feat
        elp = _DOT(feat, al_ref[...])
        erp = _DOT(feat, ar_ref[...])
        elp_ref[...] = elp
        erp_ref[...] = erp

        @pl.when(pl.program_id(0) == 0)
        def _():
            gmax_ref[...] = jnp.full((1, 16), -1e30, jnp.float32)

        gmax_ref[...] = jnp.maximum(gmax_ref[...],
                                    jnp.max(elp, axis=0, keepdims=True))

    return pl.pallas_call(
        body,
        grid=(_NBLK,),
        in_specs=[
            pl.BlockSpec((_RB, F), lambda g: (g, 0)),
            pl.BlockSpec((_RB, F), lambda g: (g, 0)),
            pl.BlockSpec((_RB, F), lambda g: (g + _NBLK, 0)),
            pl.BlockSpec((F, F), lambda g: (0, 0)),
            pl.BlockSpec((F, 16), lambda g: (0, 0)),
            pl.BlockSpec((F, 16), lambda g: (0, 0)),
        ],
        out_specs=[
            pl.BlockSpec((_RB, F), lambda g: (g, 0)),
            pl.BlockSpec((_RB, 16), lambda g: (g, 0)),
            pl.BlockSpec((_RB, 16), lambda g: (g, 0)),
            pl.BlockSpec((1, 16), lambda g: (0, 0)),
        ],
        out_shape=[
            jax.ShapeDtypeStruct((NP, F), jnp.float32),
            jax.ShapeDtypeStruct((NP, 16), jnp.float32),
            jax.ShapeDtypeStruct((NP, 16), jnp.float32),
            jax.ShapeDtypeStruct((1, 16), jnp.float32),
        ],
    )(hbase, agg2, agg2, W, Al16, Ar16)


def _tc_epilogue(msg2, den2, Rmat, bias, base, residual_relu):
    def body(m0_ref, m1_ref, d0_ref, d1_ref, r_ref, b_ref, base_ref, o_ref):
        den = d0_ref[...] + d1_ref[...]
        den_bc = _DOT(den[:, 0:8], r_ref[...])
        out = (m0_ref[...] + m1_ref[...]) / jnp.maximum(den_bc, 1e-30)
        out = out + b_ref[...]
        if residual_relu:
            out = jnp.maximum(out + base_ref[...], 0.0)
        o_ref[...] = out

    return pl.pallas_call(
        body,
        grid=(_NBLK,),
        in_specs=[
            pl.BlockSpec((_RB, F), lambda g: (g, 0)),
            pl.BlockSpec((_RB, F), lambda g: (g + _NBLK, 0)),
            pl.BlockSpec((_RB, 16), lambda g: (g, 0)),
            pl.BlockSpec((_RB, 16), lambda g: (g + _NBLK, 0)),
            pl.BlockSpec((8, F), lambda g: (0, 0)),
            pl.BlockSpec((1, F), lambda g: (0, 0)),
            pl.BlockSpec((_RB, F), lambda g: (g, 0)),
        ],
        out_specs=pl.BlockSpec((_RB, F), lambda g: (g, 0)),
        out_shape=jax.ShapeDtypeStruct((NP, F), jnp.float32),
    )(msg2, msg2, den2, den2, Rmat, bias, base)


def _attn_mat(a):
    m = jnp.zeros((H, D, 16), jnp.float32)
    m = m + a.reshape(H, D, 1) * jax.nn.one_hot(jnp.arange(H), 16,
                                               dtype=jnp.float32)[:, None, :]
    return m.reshape(F, 16)


def kernel(in_feat, edge_index, edge_weights, W0, al0, ar0, b0, W1, al1, ar1, b1):
    src = edge_index[0].astype(jnp.int32)
    dst = edge_index[1].astype(jnp.int32)
    w = edge_weights.astype(jnp.float32)

    Al0, Ar0 = _attn_mat(al0), _attn_mat(ar0)
    Al1, Ar1 = _attn_mat(al1), _attn_mat(ar1)
    Rmat = jnp.kron(jnp.eye(8, dtype=jnp.float32), jnp.ones((1, 16), jnp.float32))
    b0r = b0.reshape(1, F)
    b1r = b1.reshape(1, F)
    hp = jnp.pad(in_feat, ((0, NP - N), (0, 0)))
    zf = jnp.zeros((NP, F), jnp.float32)
    zd = jnp.zeros((NP, 16), jnp.float32)

    agg0 = _sc_weighted_agg(hp, src, dst, w, zf)
    feat0, elp0, erp0, gmax0 = _tc_feat(hp, agg0, W0, Al0, Ar0)
    msg0, den0 = _sc_attn_msg(feat0, elp0, erp0, gmax0, src, dst, zf, zd)
    h = _tc_epilogue(msg0, den0, Rmat, b0r, hp, True)

    agg1 = _sc_weighted_agg(h, src, dst, w, zf)
    feat1, elp1, erp1, gmax1 = _tc_feat(h, agg1, W1, Al1, Ar1)
    msg1, den1 = _sc_attn_msg(feat1, elp1, erp1, gmax1, src, dst, zf, zd)
    out = _tc_epilogue(msg1, den1, Rmat, b1r, h, False)
    return out[:N]
